# baseline (speedup 1.0000x reference)
import jax
import jax.numpy as jnp
from jax.experimental import pallas as pl
from jax.experimental.pallas import tpu as pltpu

B, S, D = 1, 2048, 2048
H, DH = 16, 64
ROT, HALF = 32, 16
TOPK = 512
NEG = float(jnp.finfo(jnp.float32).min)
BF = jnp.bfloat16

ROWS_SCORE = 256
ROWS_SORT = 64


def _proj_kernel(wqT_ref, wkT_ref, wwT_ref, xT_ref, qT_ref, kT_ref, wT_ref):
    xb = xT_ref[:].astype(BF)
    dims = (((1,), (0,)), ((), ()))
    qT_ref[:] = jax.lax.dot_general(
        wqT_ref[:].astype(BF), xb, dims, preferred_element_type=jnp.float32)
    kT_ref[:] = jax.lax.dot_general(
        wkT_ref[:].astype(BF), xb, dims, preferred_element_type=jnp.float32)
    wT_ref[:] = jax.lax.dot_general(
        wwT_ref[:].astype(BF), xb, dims, preferred_element_type=jnp.float32)


def _rope_cols(v, cos, sin):
    v1 = v[:HALF, :]
    v2 = v[HALF:ROT, :]
    o1 = v1 * cos - v2 * sin
    o2 = v2 * cos + v1 * sin
    return jnp.concatenate([o1, o2, v[ROT:, :]], axis=0)


def _score_kernel(qT_ref, kT_ref, w_ref, cosT_ref, sinT_ref, cosTb_ref,
                  sinTb_ref, s_ref):
    ri = pl.program_id(0)
    cosb = cosTb_ref[:]
    sinb = sinTb_ref[:]
    kb = _rope_cols(kT_ref[:], cosT_ref[:], sinT_ref[:]).astype(BF)
    w = w_ref[:]
    r = w.shape[0]
    acc = jnp.zeros((r, S), dtype=jnp.float32)
    for h in range(H):
        qhT = _rope_cols(qT_ref[h * DH:(h + 1) * DH, :], cosb, sinb)
        sh = jax.lax.dot_general(
            qhT.astype(BF), kb, (((0,), (0,)), ((), ())),
            preferred_element_type=jnp.float32)
        sb = jnp.maximum(sh, 0.0).astype(BF).astype(jnp.float32)
        wh = w[:, h:h + 1].astype(BF).astype(jnp.float32)
        acc = acc + wh * sb
    acc = acc * 0.125
    col = jax.lax.broadcasted_iota(jnp.int32, (r, S), 1)
    row = ri * ROWS_SCORE + jax.lax.broadcasted_iota(jnp.int32, (r, S), 0)
    s_ref[:] = jnp.where(col <= row, acc, NEG)


def _sort_kernel(s_ref, vals_ref, idx_ref):
    key = s_ref[:]
    r = key.shape[0]
    col = jax.lax.broadcasted_iota(jnp.int32, (r, S), 1)
    idx = col

    def stage(key, idx, kk, j):
        upper = (col & j) != 0
        pk = jnp.where(upper, pltpu.roll(key, j, axis=1),
                       pltpu.roll(key, S - j, axis=1))
        pi = jnp.where(upper, pltpu.roll(idx, j, axis=1),
                       pltpu.roll(idx, S - j, axis=1))
        up = (col & kk) == 0
        want_before = jnp.logical_xor(up, upper)
        self_before = (key > pk) | ((key == pk) & (idx < pi))
        take_self = want_before == self_before
        return (jnp.where(take_self, key, pk),
                jnp.where(take_self, idx, pi))

    kk = 2
    while kk <= S:
        nj = kk.bit_length() - 1

        def body(t, carry, kk=kk):
            kcur, icur = carry
            j = jnp.int32(kk) >> (t + 1)
            return stage(kcur, icur, jnp.int32(kk), j)

        key, idx = jax.lax.fori_loop(0, nj, body, (key, idx))
        kk *= 2

    vals_ref[:] = key[:, :TOPK]
    idx_ref[:] = idx[:, :TOPK]


def _rope_cache():
    inv_freq = 1.0 / (10000.0 ** (jnp.arange(HALF, dtype=jnp.float32) / HALF))
    ang = jnp.arange(S, dtype=jnp.float32)[:, None] * inv_freq[None, :]
    return jnp.cos(ang), jnp.sin(ang)


@jax.jit
def kernel(x, Wq, Wk, Ww):
    xT = jnp.transpose(x[0])
    cos, sin = _rope_cache()
    cosT = jnp.transpose(cos)
    sinT = jnp.transpose(sin)

    qT, kT, wT = pl.pallas_call(
        _proj_kernel,
        out_shape=[
            jax.ShapeDtypeStruct((H * DH, S), jnp.float32),
            jax.ShapeDtypeStruct((DH, S), jnp.float32),
            jax.ShapeDtypeStruct((H, S), jnp.float32),
        ],
    )(jnp.transpose(Wq), jnp.transpose(Wk), jnp.transpose(Ww), xT)
    w = jnp.transpose(wT)

    n_score = S // ROWS_SCORE
    scores = pl.pallas_call(
        _score_kernel,
        grid=(n_score,),
        in_specs=[
            pl.BlockSpec((H * DH, ROWS_SCORE), lambda i: (0, i)),
            pl.BlockSpec((DH, S), lambda i: (0, 0)),
            pl.BlockSpec((ROWS_SCORE, H), lambda i: (i, 0)),
            pl.BlockSpec((HALF, S), lambda i: (0, 0)),
            pl.BlockSpec((HALF, S), lambda i: (0, 0)),
            pl.BlockSpec((HALF, ROWS_SCORE), lambda i: (0, i)),
            pl.BlockSpec((HALF, ROWS_SCORE), lambda i: (0, i)),
        ],
        out_specs=pl.BlockSpec((ROWS_SCORE, S), lambda i: (i, 0)),
        out_shape=jax.ShapeDtypeStruct((S, S), jnp.float32),
        compiler_params=pltpu.CompilerParams(
            dimension_semantics=("arbitrary",)),
    )(qT, kT, w, cosT, sinT, cosT, sinT)

    n_sort = S // ROWS_SORT
    vals, idx = pl.pallas_call(
        _sort_kernel,
        grid=(n_sort,),
        in_specs=[pl.BlockSpec((ROWS_SORT, S), lambda i: (i, 0))],
        out_specs=[
            pl.BlockSpec((ROWS_SORT, TOPK), lambda i: (i, 0)),
            pl.BlockSpec((ROWS_SORT, TOPK), lambda i: (i, 0)),
        ],
        out_shape=[
            jax.ShapeDtypeStruct((S, TOPK), jnp.float32),
            jax.ShapeDtypeStruct((S, TOPK), jnp.int32),
        ],
        compiler_params=pltpu.CompilerParams(
            dimension_semantics=("arbitrary",)),
    )(scores)

    return vals[None], idx[None]


# speedup vs baseline: 1.5954x; 1.5954x over previous
import jax
import jax.numpy as jnp
from jax.experimental import pallas as pl
from jax.experimental.pallas import tpu as pltpu

B, S, D = 1, 2048, 2048
H, DH = 16, 64
ROT, HALF = 32, 16
TOPK = 512
NEG = float(jnp.finfo(jnp.float32).min)
BF = jnp.bfloat16

ROWS_SCORE = 256
ROWS_SORT = 64


def _proj_kernel(wqT_ref, wkT_ref, wwT_ref, xT_ref, qT_ref, kT_ref, wT_ref):
    xb = xT_ref[:].astype(BF)
    dims = (((1,), (0,)), ((), ()))
    qT_ref[:] = jax.lax.dot_general(
        wqT_ref[:].astype(BF), xb, dims, preferred_element_type=jnp.float32)
    kT_ref[:] = jax.lax.dot_general(
        wkT_ref[:].astype(BF), xb, dims, preferred_element_type=jnp.float32)
    wT_ref[:] = jax.lax.dot_general(
        wwT_ref[:].astype(BF), xb, dims, preferred_element_type=jnp.float32)


def _rope_cols(v, cos, sin):
    v1 = v[:HALF, :]
    v2 = v[HALF:ROT, :]
    o1 = v1 * cos - v2 * sin
    o2 = v2 * cos + v1 * sin
    return jnp.concatenate([o1, o2, v[ROT:, :]], axis=0)


def _score_kernel(qT_ref, kT_ref, w_ref, cosT_ref, sinT_ref, cosTb_ref,
                  sinTb_ref, s_ref):
    ri = pl.program_id(0)
    cosb = cosTb_ref[:]
    sinb = sinTb_ref[:]
    kb = _rope_cols(kT_ref[:], cosT_ref[:], sinT_ref[:]).astype(BF)
    w = w_ref[:]
    r = w.shape[0]
    acc = jnp.zeros((r, S), dtype=jnp.float32)
    for h in range(H):
        qhT = _rope_cols(qT_ref[h * DH:(h + 1) * DH, :], cosb, sinb)
        sh = jax.lax.dot_general(
            qhT.astype(BF), kb, (((0,), (0,)), ((), ())),
            preferred_element_type=jnp.float32)
        sb = jnp.maximum(sh, 0.0).astype(BF).astype(jnp.float32)
        wh = w[:, h:h + 1].astype(BF).astype(jnp.float32)
        acc = acc + wh * sb
    acc = acc * 0.125
    col = jax.lax.broadcasted_iota(jnp.int32, (r, S), 1)
    row = ri * ROWS_SCORE + jax.lax.broadcasted_iota(jnp.int32, (r, S), 0)
    s_ref[:] = jnp.where(col <= row, acc, NEG)


def _make_sort_kernel(width):
    def _sort_kernel(s_ref, vals_ref, idx_ref):
        key = s_ref[:]
        r = key.shape[0]
        col = jax.lax.broadcasted_iota(jnp.int32, (r, width), 1)
        idx = col

        def stage(key, idx, kk, j):
            upper = (col & j) != 0
            pk = jnp.where(upper, pltpu.roll(key, j, axis=1),
                           pltpu.roll(key, width - j, axis=1))
            pi = jnp.where(upper, pltpu.roll(idx, j, axis=1),
                           pltpu.roll(idx, width - j, axis=1))
            up = (col & kk) == 0
            want_before = jnp.logical_xor(up, upper)
            self_before = (key > pk) | ((key == pk) & (idx < pi))
            take_self = want_before == self_before
            return (jnp.where(take_self, key, pk),
                    jnp.where(take_self, idx, pi))

        kk = 2
        while kk <= width:
            nj = kk.bit_length() - 1

            def body(t, carry, kk=kk):
                kcur, icur = carry
                j = jnp.int32(kk) >> (t + 1)
                return stage(kcur, icur, jnp.int32(kk), j)

            key, idx = jax.lax.fori_loop(0, nj, body, (key, idx))
            kk *= 2

        if width >= TOPK:
            vals_ref[:] = key[:, :TOPK]
            idx_ref[:] = idx[:, :TOPK]
        else:
            pad = TOPK - width
            tail_i = width + jax.lax.broadcasted_iota(jnp.int32, (r, pad), 1)
            vals_ref[:] = jnp.concatenate(
                [key, jnp.full((r, pad), NEG, jnp.float32)], axis=1)
            idx_ref[:] = jnp.concatenate([idx, tail_i], axis=1)

    return _sort_kernel


def _sort_rows(scores, row_lo, row_hi, width):
    rows = row_hi - row_lo
    n_sort = rows // ROWS_SORT
    off = row_lo // ROWS_SORT
    return pl.pallas_call(
        _make_sort_kernel(width),
        grid=(n_sort,),
        in_specs=[pl.BlockSpec((ROWS_SORT, width), lambda i, off=off: (i + off, 0))],
        out_specs=[
            pl.BlockSpec((ROWS_SORT, TOPK), lambda i: (i, 0)),
            pl.BlockSpec((ROWS_SORT, TOPK), lambda i: (i, 0)),
        ],
        out_shape=[
            jax.ShapeDtypeStruct((rows, TOPK), jnp.float32),
            jax.ShapeDtypeStruct((rows, TOPK), jnp.int32),
        ],
        compiler_params=pltpu.CompilerParams(
            dimension_semantics=("arbitrary",)),
    )(scores)


def _rope_cache():
    inv_freq = 1.0 / (10000.0 ** (jnp.arange(HALF, dtype=jnp.float32) / HALF))
    ang = jnp.arange(S, dtype=jnp.float32)[:, None] * inv_freq[None, :]
    return jnp.cos(ang), jnp.sin(ang)


@jax.jit
def kernel(x, Wq, Wk, Ww):
    xT = jnp.transpose(x[0])
    cos, sin = _rope_cache()
    cosT = jnp.transpose(cos)
    sinT = jnp.transpose(sin)

    qT, kT, wT = pl.pallas_call(
        _proj_kernel,
        out_shape=[
            jax.ShapeDtypeStruct((H * DH, S), jnp.float32),
            jax.ShapeDtypeStruct((DH, S), jnp.float32),
            jax.ShapeDtypeStruct((H, S), jnp.float32),
        ],
    )(jnp.transpose(Wq), jnp.transpose(Wk), jnp.transpose(Ww), xT)
    w = jnp.transpose(wT)

    n_score = S // ROWS_SCORE
    scores = pl.pallas_call(
        _score_kernel,
        grid=(n_score,),
        in_specs=[
            pl.BlockSpec((H * DH, ROWS_SCORE), lambda i: (0, i)),
            pl.BlockSpec((DH, S), lambda i: (0, 0)),
            pl.BlockSpec((ROWS_SCORE, H), lambda i: (i, 0)),
            pl.BlockSpec((HALF, S), lambda i: (0, 0)),
            pl.BlockSpec((HALF, S), lambda i: (0, 0)),
            pl.BlockSpec((HALF, ROWS_SCORE), lambda i: (0, i)),
            pl.BlockSpec((HALF, ROWS_SCORE), lambda i: (0, i)),
        ],
        out_specs=pl.BlockSpec((ROWS_SCORE, S), lambda i: (i, 0)),
        out_shape=jax.ShapeDtypeStruct((S, S), jnp.float32),
        compiler_params=pltpu.CompilerParams(
            dimension_semantics=("arbitrary",)),
    )(qT, kT, w, cosT, sinT, cosT, sinT)

    parts = [
        _sort_rows(scores, 0, 256, 256),
        _sort_rows(scores, 256, 512, 512),
        _sort_rows(scores, 512, 1024, 1024),
        _sort_rows(scores, 1024, 2048, 2048),
    ]
    vals = jnp.concatenate([p[0] for p in parts], axis=0)
    idx = jnp.concatenate([p[1] for p in parts], axis=0)

    return vals[None], idx[None]


# speedup vs baseline: 1.5970x; 1.0010x over previous
import jax
import jax.numpy as jnp
from jax.experimental import pallas as pl
from jax.experimental.pallas import tpu as pltpu

B, S, D = 1, 2048, 2048
H, DH = 16, 64
ROT, HALF = 32, 16
TOPK = 512
NEG = float(jnp.finfo(jnp.float32).min)
BF = jnp.bfloat16

ROWS_SCORE = 256
ROWS_SORT = 64


def _proj_kernel(wqT_ref, wkT_ref, wwT_ref, xT_ref, qT_ref, kT_ref, wT_ref):
    xb = xT_ref[:].astype(BF)
    dims = (((1,), (0,)), ((), ()))
    qT_ref[:] = jax.lax.dot_general(
        wqT_ref[:].astype(BF), xb, dims, preferred_element_type=jnp.float32)
    kT_ref[:] = jax.lax.dot_general(
        wkT_ref[:].astype(BF), xb, dims, preferred_element_type=jnp.float32)
    wT_ref[:] = jax.lax.dot_general(
        wwT_ref[:].astype(BF), xb, dims, preferred_element_type=jnp.float32)


def _rope_cols(v, cos, sin):
    v1 = v[:HALF, :]
    v2 = v[HALF:ROT, :]
    o1 = v1 * cos - v2 * sin
    o2 = v2 * cos + v1 * sin
    return jnp.concatenate([o1, o2, v[ROT:, :]], axis=0)


def _score_kernel(qT_ref, kT_ref, w_ref, cosT_ref, sinT_ref, cosTb_ref,
                  sinTb_ref, s_ref):
    ri = pl.program_id(0)
    cosb = cosTb_ref[:]
    sinb = sinTb_ref[:]
    kb = _rope_cols(kT_ref[:], cosT_ref[:], sinT_ref[:]).astype(BF)
    w = w_ref[:]
    r = w.shape[0]
    acc = jnp.zeros((r, S), dtype=jnp.float32)
    for h in range(H):
        qhT = _rope_cols(qT_ref[h * DH:(h + 1) * DH, :], cosb, sinb)
        sh = jax.lax.dot_general(
            qhT.astype(BF), kb, (((0,), (0,)), ((), ())),
            preferred_element_type=jnp.float32)
        sb = jnp.maximum(sh, 0.0).astype(BF).astype(jnp.float32)
        wh = w[:, h:h + 1].astype(BF).astype(jnp.float32)
        acc = acc + wh * sb
    acc = acc * 0.125
    col = jax.lax.broadcasted_iota(jnp.int32, (r, S), 1)
    row = ri * ROWS_SCORE + jax.lax.broadcasted_iota(jnp.int32, (r, S), 0)
    s_ref[:] = jnp.where(col <= row, acc, NEG)


def _make_sort_kernel(width):
    def _sort_kernel(s_ref, vals_ref, idx_ref):
        key = s_ref[:]
        r = key.shape[0]
        col = jax.lax.broadcasted_iota(jnp.int32, (r, width), 1)
        idx = col

        def stage(key, idx, kk, j):
            upper = (col & j) != 0
            pk = jnp.where(upper, pltpu.roll(key, j, axis=1),
                           pltpu.roll(key, width - j, axis=1))
            pi = jnp.where(upper, pltpu.roll(idx, j, axis=1),
                           pltpu.roll(idx, width - j, axis=1))
            up = (col & kk) == 0
            want_before = jnp.logical_xor(up, upper)
            self_before = (key > pk) | ((key == pk) & (idx < pi))
            take_self = want_before == self_before
            return (jnp.where(take_self, key, pk),
                    jnp.where(take_self, idx, pi))

        kk = 2
        while kk <= width:
            nj = kk.bit_length() - 1

            def body(t, carry, kk=kk):
                kcur, icur = carry
                j = jnp.int32(kk) >> (t + 1)
                return stage(kcur, icur, jnp.int32(kk), j)

            key, idx = jax.lax.fori_loop(0, nj, body, (key, idx))
            kk *= 2

        if width >= TOPK:
            vals_ref[:] = key[:, :TOPK]
            idx_ref[:] = idx[:, :TOPK]
        else:
            pad = TOPK - width
            tail_i = width + jax.lax.broadcasted_iota(jnp.int32, (r, pad), 1)
            vals_ref[:] = jnp.concatenate(
                [key, jnp.full((r, pad), NEG, jnp.float32)], axis=1)
            idx_ref[:] = jnp.concatenate([idx, tail_i], axis=1)

    return _sort_kernel


def _sort_rows(scores, row_lo, row_hi, width):
    rows = row_hi - row_lo
    n_sort = rows // ROWS_SORT
    off = row_lo // ROWS_SORT
    return pl.pallas_call(
        _make_sort_kernel(width),
        grid=(n_sort,),
        in_specs=[pl.BlockSpec((ROWS_SORT, width), lambda i, off=off: (i + off, 0))],
        out_specs=[
            pl.BlockSpec((ROWS_SORT, TOPK), lambda i: (i, 0)),
            pl.BlockSpec((ROWS_SORT, TOPK), lambda i: (i, 0)),
        ],
        out_shape=[
            jax.ShapeDtypeStruct((rows, TOPK), jnp.float32),
            jax.ShapeDtypeStruct((rows, TOPK), jnp.int32),
        ],
        compiler_params=pltpu.CompilerParams(
            dimension_semantics=("parallel",)),
    )(scores)


def _rope_cache():
    inv_freq = 1.0 / (10000.0 ** (jnp.arange(HALF, dtype=jnp.float32) / HALF))
    ang = jnp.arange(S, dtype=jnp.float32)[:, None] * inv_freq[None, :]
    return jnp.cos(ang), jnp.sin(ang)


@jax.jit
def kernel(x, Wq, Wk, Ww):
    xT = jnp.transpose(x[0])
    cos, sin = _rope_cache()
    cosT = jnp.transpose(cos)
    sinT = jnp.transpose(sin)

    qT, kT, wT = pl.pallas_call(
        _proj_kernel,
        out_shape=[
            jax.ShapeDtypeStruct((H * DH, S), jnp.float32),
            jax.ShapeDtypeStruct((DH, S), jnp.float32),
            jax.ShapeDtypeStruct((H, S), jnp.float32),
        ],
    )(jnp.transpose(Wq), jnp.transpose(Wk), jnp.transpose(Ww), xT)
    w = jnp.transpose(wT)

    n_score = S // ROWS_SCORE
    scores = pl.pallas_call(
        _score_kernel,
        grid=(n_score,),
        in_specs=[
            pl.BlockSpec((H * DH, ROWS_SCORE), lambda i: (0, i)),
            pl.BlockSpec((DH, S), lambda i: (0, 0)),
            pl.BlockSpec((ROWS_SCORE, H), lambda i: (i, 0)),
            pl.BlockSpec((HALF, S), lambda i: (0, 0)),
            pl.BlockSpec((HALF, S), lambda i: (0, 0)),
            pl.BlockSpec((HALF, ROWS_SCORE), lambda i: (0, i)),
            pl.BlockSpec((HALF, ROWS_SCORE), lambda i: (0, i)),
        ],
        out_specs=pl.BlockSpec((ROWS_SCORE, S), lambda i: (i, 0)),
        out_shape=jax.ShapeDtypeStruct((S, S), jnp.float32),
        compiler_params=pltpu.CompilerParams(
            dimension_semantics=("parallel",)),
    )(qT, kT, w, cosT, sinT, cosT, sinT)

    parts = [
        _sort_rows(scores, 0, 256, 256),
        _sort_rows(scores, 256, 512, 512),
        _sort_rows(scores, 512, 1024, 1024),
        _sort_rows(scores, 1024, 2048, 2048),
    ]
    vals = jnp.concatenate([p[0] for p in parts], axis=0)
    idx = jnp.concatenate([p[1] for p in parts], axis=0)

    return vals[None], idx[None]


# speedup vs baseline: 1.9973x; 1.2507x over previous
import jax
import jax.numpy as jnp
from jax.experimental import pallas as pl
from jax.experimental.pallas import tpu as pltpu

B, S, D = 1, 2048, 2048
H, DH = 16, 64
ROT, HALF = 32, 16
TOPK = 512
NEG = float(jnp.finfo(jnp.float32).min)
BF = jnp.bfloat16

ROWS_SCORE = 256
ROWS_SORT = 64


def _proj_kernel(wqT_ref, wkT_ref, wwT_ref, xT_ref, qT_ref, kT_ref, wT_ref):
    xb = xT_ref[:].astype(BF)
    dims = (((1,), (0,)), ((), ()))
    qT_ref[:] = jax.lax.dot_general(
        wqT_ref[:].astype(BF), xb, dims, preferred_element_type=jnp.float32)
    kT_ref[:] = jax.lax.dot_general(
        wkT_ref[:].astype(BF), xb, dims, preferred_element_type=jnp.float32)
    wT_ref[:] = jax.lax.dot_general(
        wwT_ref[:].astype(BF), xb, dims, preferred_element_type=jnp.float32)


def _rope_cols(v, cos, sin):
    v1 = v[:HALF, :]
    v2 = v[HALF:ROT, :]
    o1 = v1 * cos - v2 * sin
    o2 = v2 * cos + v1 * sin
    return jnp.concatenate([o1, o2, v[ROT:, :]], axis=0)


def _score_kernel(qT_ref, kT_ref, w_ref, cosT_ref, sinT_ref, cosTb_ref,
                  sinTb_ref, s_ref):
    ri = pl.program_id(0)
    cosb = cosTb_ref[:]
    sinb = sinTb_ref[:]
    kb = _rope_cols(kT_ref[:], cosT_ref[:], sinT_ref[:]).astype(BF)
    w = w_ref[:]
    r = w.shape[0]
    acc = jnp.zeros((r, S), dtype=jnp.float32)
    for h in range(H):
        qhT = _rope_cols(qT_ref[h * DH:(h + 1) * DH, :], cosb, sinb)
        sh = jax.lax.dot_general(
            qhT.astype(BF), kb, (((0,), (0,)), ((), ())),
            preferred_element_type=jnp.float32)
        sb = jnp.maximum(sh, 0.0).astype(BF).astype(jnp.float32)
        wh = w[:, h:h + 1].astype(BF).astype(jnp.float32)
        acc = acc + wh * sb
    acc = acc * 0.125
    col = jax.lax.broadcasted_iota(jnp.int32, (r, S), 1)
    row = ri * ROWS_SCORE + jax.lax.broadcasted_iota(jnp.int32, (r, S), 0)
    s_ref[:] = jnp.where(col <= row, acc, NEG)


def _make_sort_kernel(width):
    def _sort_kernel(s_ref, vals_ref, idx_ref):
        key = s_ref[:]
        r = key.shape[0]
        idx = jax.lax.broadcasted_iota(jnp.int32, (r, width), 1)

        def stage(key, idx, kk, j, wcur):
            col = jax.lax.broadcasted_iota(jnp.int32, (r, wcur), 1)
            upper = (col & j) != 0
            pk = jnp.where(upper, pltpu.roll(key, j, axis=1),
                           pltpu.roll(key, wcur - j, axis=1))
            pi = jnp.where(upper, pltpu.roll(idx, j, axis=1),
                           pltpu.roll(idx, wcur - j, axis=1))
            up = (col & kk) == 0
            want_before = jnp.logical_xor(up, upper)
            self_before = (key > pk) | ((key == pk) & (idx < pi))
            take_self = want_before == self_before
            return (jnp.where(take_self, key, pk),
                    jnp.where(take_self, idx, pi))

        def phase(key, idx, kk, wcur, j_hi):
            nj = j_hi.bit_length()

            def body(t, carry, kk=kk, wcur=wcur, j_hi=j_hi):
                kcur, icur = carry
                j = jnp.int32(j_hi) >> t
                return stage(kcur, icur, jnp.int32(kk), j, wcur)

            return jax.lax.fori_loop(0, nj, body, (key, idx))

        kk = 2
        while kk <= min(width, TOPK):
            key, idx = phase(key, idx, kk, width, kk // 2)
            kk *= 2

        if width == 4 * TOPK:
            key, idx = stage(key, idx, 2 * TOPK, TOPK, width)
            key = jnp.concatenate([key[:, :TOPK], key[:, 3 * TOPK:]], axis=1)
            idx = jnp.concatenate([idx[:, :TOPK], idx[:, 3 * TOPK:]], axis=1)
            key, idx = phase(key, idx, TOPK, 2 * TOPK, TOPK // 2)
        if width > TOPK:
            key, idx = stage(key, idx, 2 * TOPK, TOPK, 2 * TOPK)
            key = key[:, :TOPK]
            idx = idx[:, :TOPK]
            key, idx = phase(key, idx, 2 * TOPK, TOPK, TOPK // 2)

        if width >= TOPK:
            vals_ref[:] = key[:, :TOPK]
            idx_ref[:] = idx[:, :TOPK]
        else:
            pad = TOPK - width
            tail_i = width + jax.lax.broadcasted_iota(jnp.int32, (r, pad), 1)
            vals_ref[:] = jnp.concatenate(
                [key, jnp.full((r, pad), NEG, jnp.float32)], axis=1)
            idx_ref[:] = jnp.concatenate([idx, tail_i], axis=1)

    return _sort_kernel


def _sort_rows(scores, row_lo, row_hi, width):
    rows = row_hi - row_lo
    n_sort = rows // ROWS_SORT
    off = row_lo // ROWS_SORT
    return pl.pallas_call(
        _make_sort_kernel(width),
        grid=(n_sort,),
        in_specs=[pl.BlockSpec((ROWS_SORT, width), lambda i, off=off: (i + off, 0))],
        out_specs=[
            pl.BlockSpec((ROWS_SORT, TOPK), lambda i: (i, 0)),
            pl.BlockSpec((ROWS_SORT, TOPK), lambda i: (i, 0)),
        ],
        out_shape=[
            jax.ShapeDtypeStruct((rows, TOPK), jnp.float32),
            jax.ShapeDtypeStruct((rows, TOPK), jnp.int32),
        ],
        compiler_params=pltpu.CompilerParams(
            dimension_semantics=("parallel",)),
    )(scores)


def _rope_cache():
    inv_freq = 1.0 / (10000.0 ** (jnp.arange(HALF, dtype=jnp.float32) / HALF))
    ang = jnp.arange(S, dtype=jnp.float32)[:, None] * inv_freq[None, :]
    return jnp.cos(ang), jnp.sin(ang)


@jax.jit
def kernel(x, Wq, Wk, Ww):
    xT = jnp.transpose(x[0])
    cos, sin = _rope_cache()
    cosT = jnp.transpose(cos)
    sinT = jnp.transpose(sin)

    qT, kT, wT = pl.pallas_call(
        _proj_kernel,
        out_shape=[
            jax.ShapeDtypeStruct((H * DH, S), jnp.float32),
            jax.ShapeDtypeStruct((DH, S), jnp.float32),
            jax.ShapeDtypeStruct((H, S), jnp.float32),
        ],
    )(jnp.transpose(Wq), jnp.transpose(Wk), jnp.transpose(Ww), xT)
    w = jnp.transpose(wT)

    n_score = S // ROWS_SCORE
    scores = pl.pallas_call(
        _score_kernel,
        grid=(n_score,),
        in_specs=[
            pl.BlockSpec((H * DH, ROWS_SCORE), lambda i: (0, i)),
            pl.BlockSpec((DH, S), lambda i: (0, 0)),
            pl.BlockSpec((ROWS_SCORE, H), lambda i: (i, 0)),
            pl.BlockSpec((HALF, S), lambda i: (0, 0)),
            pl.BlockSpec((HALF, S), lambda i: (0, 0)),
            pl.BlockSpec((HALF, ROWS_SCORE), lambda i: (0, i)),
            pl.BlockSpec((HALF, ROWS_SCORE), lambda i: (0, i)),
        ],
        out_specs=pl.BlockSpec((ROWS_SCORE, S), lambda i: (i, 0)),
        out_shape=jax.ShapeDtypeStruct((S, S), jnp.float32),
        compiler_params=pltpu.CompilerParams(
            dimension_semantics=("parallel",)),
    )(qT, kT, w, cosT, sinT, cosT, sinT)

    parts = [
        _sort_rows(scores, 0, 256, 256),
        _sort_rows(scores, 256, 512, 512),
        _sort_rows(scores, 512, 1024, 1024),
        _sort_rows(scores, 1024, 2048, 2048),
    ]
    vals = jnp.concatenate([p[0] for p in parts], axis=0)
    idx = jnp.concatenate([p[1] for p in parts], axis=0)

    return vals[None], idx[None]
